# baseline (speedup 1.0000x reference)
import jax
import jax.numpy as jnp
from jax.experimental import pallas as pl
from jax.experimental.pallas import tpu as pltpu

HIDDEN = 768
NUM_PROPS = 32
NUM_EXPERTS = 8
BATCH = 2048
OUT_DIM = NUM_PROPS * 2
BB = 1024
NB = BATCH // BB

_HI = jax.lax.Precision.HIGHEST


def _dot(a, b):
    return jax.lax.dot_general(
        a.astype(jnp.bfloat16), b, (((1,), (0,)), ((), ())),
        preferred_element_type=jnp.float32)


def _moe_body(xq_ref, rW1_ref, rb1_ref, rW2_ref, rb2_ref,
              eW1_ref, eb1_ref, eW2_ref, eb2_ref, out_ref, comb_ref, acc_ref):
    e = pl.program_id(1)
    col = jax.lax.broadcasted_iota(jnp.int32, (BB, NUM_EXPERTS), 1)

    @pl.when(e == 0)
    def _router():
        h = jnp.maximum(_dot(xq_ref[...], rW1_ref[...]) + rb1_ref[...], 0.0)
        logits = _dot(h, rW2_ref[...]) + rb2_ref[...]
        m = jnp.max(logits, axis=-1, keepdims=True)
        ex = jnp.exp(logits - m)
        p = ex / jnp.sum(ex, axis=-1, keepdims=True)
        w1 = jnp.max(p, axis=-1, keepdims=True)
        c1 = jnp.min(jnp.where(p >= w1, col, NUM_EXPERTS), axis=-1, keepdims=True)
        oh1 = col == c1
        pm = jnp.where(oh1, -jnp.inf, p)
        w2 = jnp.max(pm, axis=-1, keepdims=True)
        c2 = jnp.min(jnp.where(pm >= w2, col, NUM_EXPERTS), axis=-1, keepdims=True)
        oh2 = col == c2
        denom = w1 + w2 + 1e-6
        comb_ref[...] = (jnp.where(oh1, w1, 0.0) + jnp.where(oh2, w2, 0.0)) / denom

    x = xq_ref[:, :HIDDEN]
    he = jnp.maximum(_dot(x, eW1_ref[0]) + eb1_ref[0], 0.0)
    o = _dot(he, eW2_ref[0]) + eb2_ref[0]
    w_col = jnp.sum(jnp.where(col == e, comb_ref[...], 0.0), axis=-1, keepdims=True)
    contrib = w_col * o

    @pl.when(e == 0)
    def _init():
        acc_ref[...] = contrib

    @pl.when(e > 0)
    def _acc():
        acc_ref[...] += contrib

    @pl.when(e == NUM_EXPERTS - 1)
    def _fin():
        out_ref[...] = jax.nn.sigmoid(acc_ref[...])


@jax.jit
def kernel(multimodal_feat, query_feat, rW1, rb1, rW2, rb2, eW1, eb1, eW2, eb2):
    bf = jnp.bfloat16
    xq = jnp.concatenate([multimodal_feat, query_feat], axis=-1).astype(bf)

    const = lambda b, e: (0, 0)
    out = pl.pallas_call(
        _moe_body,
        grid=(NB, NUM_EXPERTS),
        in_specs=[
            pl.BlockSpec((BB, 2 * HIDDEN), lambda b, e: (b, 0)),
            pl.BlockSpec((2 * HIDDEN, HIDDEN), const),
            pl.BlockSpec((1, HIDDEN), const),
            pl.BlockSpec((HIDDEN, NUM_EXPERTS), const),
            pl.BlockSpec((1, NUM_EXPERTS), const),
            pl.BlockSpec((1, HIDDEN, HIDDEN), lambda b, e: (e, 0, 0)),
            pl.BlockSpec((1, 1, HIDDEN), lambda b, e: (e, 0, 0)),
            pl.BlockSpec((1, HIDDEN, OUT_DIM), lambda b, e: (e, 0, 0)),
            pl.BlockSpec((1, 1, OUT_DIM), lambda b, e: (e, 0, 0)),
        ],
        out_specs=pl.BlockSpec((BB, OUT_DIM), lambda b, e: (b, 0)),
        out_shape=jax.ShapeDtypeStruct((BATCH, OUT_DIM), jnp.float32),
        scratch_shapes=[
            pltpu.VMEM((BB, NUM_EXPERTS), jnp.float32),
            pltpu.VMEM((BB, OUT_DIM), jnp.float32),
        ],
        compiler_params=pltpu.CompilerParams(
            dimension_semantics=("arbitrary", "arbitrary"),
        ),
    )(xq, rW1.astype(bf), rb1.reshape(1, HIDDEN), rW2.astype(bf),
      rb2.reshape(1, NUM_EXPERTS), eW1.astype(bf),
      eb1.reshape(NUM_EXPERTS, 1, HIDDEN), eW2.astype(bf),
      eb2.reshape(NUM_EXPERTS, 1, OUT_DIM))
    return out.reshape(BATCH * NUM_PROPS, 2)


# speedup vs baseline: 1.1773x; 1.1773x over previous
import jax
import jax.numpy as jnp
from jax.experimental import pallas as pl
from jax.experimental.pallas import tpu as pltpu

HIDDEN = 768
NUM_PROPS = 32
NUM_EXPERTS = 8
BATCH = 2048
OUT_DIM = NUM_PROPS * 2


def _dot(a, b):
    return jax.lax.dot_general(
        a.astype(jnp.bfloat16), b.astype(jnp.bfloat16), (((1,), (0,)), ((), ())),
        preferred_element_type=jnp.float32)


def _moe_body(mm_ref, qf_ref, rW1_ref, rb1_ref, rW2_ref, rb2_ref,
              eW1_ref, eb1_ref, eW2_ref, eb2_ref, out_ref, comb_ref, acc_ref):
    e = pl.program_id(0)
    col = jax.lax.broadcasted_iota(jnp.int32, (BATCH, NUM_EXPERTS), 1)

    @pl.when(e == 0)
    def _router():
        h = (_dot(mm_ref[...], rW1_ref[:HIDDEN]) +
             _dot(qf_ref[...], rW1_ref[HIDDEN:]))
        h = jnp.maximum(h + rb1_ref[...], 0.0)
        logits = _dot(h, rW2_ref[...]) + rb2_ref[...]
        m = jnp.max(logits, axis=-1, keepdims=True)
        ex = jnp.exp(logits - m)
        p = ex / jnp.sum(ex, axis=-1, keepdims=True)
        w1 = jnp.max(p, axis=-1, keepdims=True)
        c1 = jnp.min(jnp.where(p >= w1, col, NUM_EXPERTS), axis=-1, keepdims=True)
        oh1 = col == c1
        pm = jnp.where(oh1, -jnp.inf, p)
        w2 = jnp.max(pm, axis=-1, keepdims=True)
        c2 = jnp.min(jnp.where(pm >= w2, col, NUM_EXPERTS), axis=-1, keepdims=True)
        oh2 = col == c2
        denom = w1 + w2 + 1e-6
        comb_ref[...] = (jnp.where(oh1, w1, 0.0) + jnp.where(oh2, w2, 0.0)) / denom

    he = jnp.maximum(_dot(mm_ref[...], eW1_ref[0]) + eb1_ref[0], 0.0)
    o = _dot(he, eW2_ref[0]) + eb2_ref[0]
    w_col = jnp.sum(jnp.where(col == e, comb_ref[...], 0.0), axis=-1, keepdims=True)
    contrib = w_col * o

    @pl.when(e == 0)
    def _init():
        acc_ref[...] = contrib

    @pl.when(e > 0)
    def _acc():
        acc_ref[...] += contrib

    @pl.when(e == NUM_EXPERTS - 1)
    def _fin():
        out_ref[...] = jax.nn.sigmoid(acc_ref[...])


@jax.jit
def kernel(multimodal_feat, query_feat, rW1, rb1, rW2, rb2, eW1, eb1, eW2, eb2):
    const = lambda e: (0, 0)
    out = pl.pallas_call(
        _moe_body,
        grid=(NUM_EXPERTS,),
        in_specs=[
            pl.BlockSpec((BATCH, HIDDEN), const),
            pl.BlockSpec((BATCH, HIDDEN), const),
            pl.BlockSpec((2 * HIDDEN, HIDDEN), const),
            pl.BlockSpec((1, HIDDEN), const),
            pl.BlockSpec((HIDDEN, NUM_EXPERTS), const),
            pl.BlockSpec((1, NUM_EXPERTS), const),
            pl.BlockSpec((1, HIDDEN, HIDDEN), lambda e: (e, 0, 0)),
            pl.BlockSpec((1, 1, HIDDEN), lambda e: (e, 0, 0)),
            pl.BlockSpec((1, HIDDEN, OUT_DIM), lambda e: (e, 0, 0)),
            pl.BlockSpec((1, 1, OUT_DIM), lambda e: (e, 0, 0)),
        ],
        out_specs=pl.BlockSpec((BATCH, OUT_DIM), const),
        out_shape=jax.ShapeDtypeStruct((BATCH, OUT_DIM), jnp.float32),
        scratch_shapes=[
            pltpu.VMEM((BATCH, NUM_EXPERTS), jnp.float32),
            pltpu.VMEM((BATCH, OUT_DIM), jnp.float32),
        ],
        compiler_params=pltpu.CompilerParams(
            dimension_semantics=("arbitrary",),
        ),
    )(multimodal_feat, query_feat, rW1, rb1.reshape(1, HIDDEN), rW2,
      rb2.reshape(1, NUM_EXPERTS), eW1,
      eb1.reshape(NUM_EXPERTS, 1, HIDDEN), eW2,
      eb2.reshape(NUM_EXPERTS, 1, OUT_DIM))
    return out.reshape(BATCH * NUM_PROPS, 2)
